# Initial kernel scaffold; baseline (speedup 1.0000x reference)
import functools

import jax
import jax.numpy as jnp
from jax import lax
from jax.experimental import pallas as pl
from jax.experimental.pallas import tpu as pltpu
from jax.experimental.pallas import tpu_sc as plsc

N_NODES = 10000
E_EDGES = 320000
D_IN = 128
H_MID = 64
D_OUT = 128

NC = 2
NS = 16
NW = NC * NS
CH = 128
K_CHUNKS = -(-E_EDGES // (NW * CH))
E_PAD = NW * CH * K_CHUNKS
R_PAD = 10016
ZR = R_PAD // NS

_sc_mesh = plsc.VectorSubcoreMesh(core_axis_name="c", subcore_axis_name="s")


@functools.partial(
    pl.kernel,
    out_type=jax.ShapeDtypeStruct((NC, R_PAD, H_MID), jnp.float32),
    mesh=_sc_mesh,
    scratch_types=[
        pltpu.VMEM((K_CHUNKS, CH), jnp.int32),
        pltpu.VMEM((K_CHUNKS, CH), jnp.int32),
        pltpu.VMEM((CH, H_MID), jnp.float32),
        pltpu.VMEM_SHARED((R_PAD, H_MID), jnp.float32),
    ],
)
def _sc_agg(y_hbm, src_hbm, dst_hbm, zero_hbm, out_hbm,
            src_v, dst_v, rows_v, acc_sh):
    cid = lax.axis_index("c")
    sid = lax.axis_index("s")
    wid = cid * NS + sid

    pltpu.sync_copy(zero_hbm.at[pl.ds(sid * ZR, ZR)], acc_sh.at[pl.ds(sid * ZR, ZR)])
    pltpu.sync_copy(src_hbm.at[wid], src_v)
    pltpu.sync_copy(dst_hbm.at[wid], dst_v)
    plsc.subcore_barrier()

    @pl.loop(0, K_CHUNKS)
    def _(j):
        pltpu.sync_copy(y_hbm.at[src_v.at[j]], rows_v)
        pltpu.sync_copy(rows_v, acc_sh.at[dst_v.at[j]], add=True)

    plsc.subcore_barrier()
    pltpu.sync_copy(acc_sh.at[pl.ds(sid * ZR, ZR)],
                    out_hbm.at[cid, pl.ds(sid * ZR, ZR)])


def _mm_body(x_ref, w_ref, o_ref):
    o_ref[...] = jnp.dot(x_ref[...], w_ref[...],
                         preferred_element_type=jnp.float32)


def _mid_body(y_ref, p_ref, b1_ref, g1_ref, bt1_ref, w2_ref, b2_ref, w3_ref,
              o_ref):
    h = (y_ref[...] + p_ref[0, :N_NODES, :] + p_ref[1, :N_NODES, :]
         + b1_ref[...])
    m = jnp.mean(h, axis=0, keepdims=True)
    c = h - m
    v = jnp.mean(c * c, axis=0, keepdims=True)
    hn = g1_ref[...] * c / jnp.sqrt(v + 1e-5) + bt1_ref[...]
    a = jnp.maximum(hn, 0.0)
    a = jnp.maximum(
        jnp.dot(a, w2_ref[...], preferred_element_type=jnp.float32)
        + b2_ref[...], 0.0)
    o_ref[...] = jnp.dot(a, w3_ref[...], preferred_element_type=jnp.float32)


def _fin_body(y_ref, p_ref, b3_ref, g3_ref, bt3_ref, w4_ref, b4_ref, o_ref):
    h = (y_ref[...] + p_ref[0, :N_NODES, :] + p_ref[1, :N_NODES, :]
         + b3_ref[...])
    m = jnp.mean(h, axis=0, keepdims=True)
    c = h - m
    v = jnp.mean(c * c, axis=0, keepdims=True)
    hn = g3_ref[...] * c / jnp.sqrt(v + 1e-5) + bt3_ref[...]
    a = jnp.maximum(hn, 0.0)
    z = (jnp.dot(a, w4_ref[...], preferred_element_type=jnp.float32)
         + b4_ref[...])
    zm = jnp.max(z, axis=1, keepdims=True)
    zs = z - zm
    o_ref[...] = zs - jnp.log(jnp.sum(jnp.exp(zs), axis=1, keepdims=True))


def kernel(x, edge_index, W1, b1, g1, bt1, W2, b2, W3, b3, g3, bt3, W4, b4):
    pad = E_PAD - E_EDGES
    src_p = jnp.concatenate(
        [edge_index[0], jnp.zeros((pad,), jnp.int32)]).reshape(NW, K_CHUNKS, CH)
    dst_p = jnp.concatenate(
        [edge_index[1], jnp.full((pad,), N_NODES, jnp.int32)]
    ).reshape(NW, K_CHUNKS, CH)
    zero_init = jnp.zeros((R_PAD, H_MID), jnp.float32)

    y1 = pl.pallas_call(
        _mm_body,
        out_shape=jax.ShapeDtypeStruct((N_NODES, H_MID), jnp.float32),
    )(x, W1)

    p1 = _sc_agg(y1, src_p, dst_p, zero_init)

    y2 = pl.pallas_call(
        _mid_body,
        out_shape=jax.ShapeDtypeStruct((N_NODES, H_MID), jnp.float32),
    )(y1, p1, b1.reshape(1, H_MID), g1.reshape(1, H_MID),
      bt1.reshape(1, H_MID), W2, b2.reshape(1, H_MID), W3)

    p2 = _sc_agg(y2, src_p, dst_p, zero_init)

    out = pl.pallas_call(
        _fin_body,
        out_shape=jax.ShapeDtypeStruct((N_NODES, D_OUT), jnp.float32),
    )(y2, p2, b3.reshape(1, H_MID), g3.reshape(1, H_MID),
      bt3.reshape(1, H_MID), W4, b4.reshape(1, D_OUT))
    return out


# speedup vs baseline: 6.3875x; 6.3875x over previous
import functools

import jax
import jax.numpy as jnp
from jax import lax
from jax.experimental import pallas as pl
from jax.experimental.pallas import tpu as pltpu
from jax.experimental.pallas import tpu_sc as plsc

N_NODES = 10000
E_EDGES = 320000
D_IN = 128
H_MID = 64
D_OUT = 128

NC = 2
NS = 16
NW = NC * NS
CH = 128
K_CHUNKS = -(-E_EDGES // (NW * CH))
E_PAD = NW * CH * K_CHUNKS
R_PAD = 10112
ZR = R_PAD // NS

@functools.cache
def _make_sc_agg():
    mesh = plsc.VectorSubcoreMesh(core_axis_name="c", subcore_axis_name="s")

    @functools.partial(
        pl.kernel,
        out_type=jax.ShapeDtypeStruct((NC, R_PAD, H_MID), jnp.float32),
        mesh=mesh,
        compiler_params=pltpu.CompilerParams(use_tc_tiling_on_sc=False),
        scratch_types=[
            pltpu.VMEM((K_CHUNKS, CH), jnp.int32),
            pltpu.VMEM((K_CHUNKS, CH), jnp.int32),
            pltpu.VMEM((CH, H_MID), jnp.float32),
            pltpu.VMEM_SHARED((R_PAD, H_MID), jnp.float32),
        ],
    )
    def _sc_agg(y_hbm, src_hbm, dst_hbm, zero_hbm, out_hbm,
                src_v, dst_v, rows_v, acc_sh):
        cid = lax.axis_index("c")
        sid = lax.axis_index("s")
        wid = cid * NS + sid

        pltpu.sync_copy(zero_hbm.at[pl.ds(sid * ZR, ZR)],
                        acc_sh.at[pl.ds(sid * ZR, ZR)])
        pltpu.sync_copy(src_hbm.at[wid], src_v)
        pltpu.sync_copy(dst_hbm.at[wid], dst_v)
        plsc.subcore_barrier()

        @pl.loop(0, K_CHUNKS)
        def _(j):
            pltpu.sync_copy(y_hbm.at[src_v.at[j]], rows_v)
            pltpu.sync_copy(rows_v, acc_sh.at[dst_v.at[j]], add=True)

        plsc.subcore_barrier()
        pltpu.sync_copy(acc_sh.at[pl.ds(sid * ZR, ZR)],
                        out_hbm.at[cid, pl.ds(sid * ZR, ZR)])

    return _sc_agg


def _mm_body(x_ref, w_ref, o_ref):
    o_ref[...] = jnp.dot(x_ref[...], w_ref[...],
                         preferred_element_type=jnp.float32)


def _mid_body(y_ref, p_ref, b1_ref, g1_ref, bt1_ref, w2_ref, b2_ref, w3_ref,
              o_ref):
    h = (y_ref[...] + p_ref[0, :N_NODES, :] + p_ref[1, :N_NODES, :]
         + b1_ref[...])
    m = jnp.mean(h, axis=0, keepdims=True)
    c = h - m
    v = jnp.mean(c * c, axis=0, keepdims=True)
    hn = g1_ref[...] * c / jnp.sqrt(v + 1e-5) + bt1_ref[...]
    a = jnp.maximum(hn, 0.0)
    a = jnp.maximum(
        jnp.dot(a, w2_ref[...], preferred_element_type=jnp.float32)
        + b2_ref[...], 0.0)
    o_ref[...] = jnp.dot(a, w3_ref[...], preferred_element_type=jnp.float32)


def _fin_body(y_ref, p_ref, b3_ref, g3_ref, bt3_ref, w4_ref, b4_ref, o_ref):
    h = (y_ref[...] + p_ref[0, :N_NODES, :] + p_ref[1, :N_NODES, :]
         + b3_ref[...])
    m = jnp.mean(h, axis=0, keepdims=True)
    c = h - m
    v = jnp.mean(c * c, axis=0, keepdims=True)
    hn = g3_ref[...] * c / jnp.sqrt(v + 1e-5) + bt3_ref[...]
    a = jnp.maximum(hn, 0.0)
    z = (jnp.dot(a, w4_ref[...], preferred_element_type=jnp.float32)
         + b4_ref[...])
    zm = jnp.max(z, axis=1, keepdims=True)
    zs = z - zm
    o_ref[...] = zs - jnp.log(jnp.sum(jnp.exp(zs), axis=1, keepdims=True))


def kernel(x, edge_index, W1, b1, g1, bt1, W2, b2, W3, b3, g3, bt3, W4, b4):
    pad = E_PAD - E_EDGES
    src_p = jnp.concatenate(
        [edge_index[0], jnp.zeros((pad,), jnp.int32)]).reshape(NW, K_CHUNKS, CH)
    dst_p = jnp.concatenate(
        [edge_index[1], jnp.full((pad,), N_NODES, jnp.int32)]
    ).reshape(NW, K_CHUNKS, CH)
    zero_init = jnp.zeros((R_PAD, H_MID), jnp.float32)

    y1 = pl.pallas_call(
        _mm_body,
        out_shape=jax.ShapeDtypeStruct((N_NODES, H_MID), jnp.float32),
    )(x, W1)

    sc_agg = _make_sc_agg()
    p1 = sc_agg(y1, src_p, dst_p, zero_init)

    y2 = pl.pallas_call(
        _mid_body,
        out_shape=jax.ShapeDtypeStruct((N_NODES, H_MID), jnp.float32),
    )(y1, p1, b1.reshape(1, H_MID), g1.reshape(1, H_MID),
      bt1.reshape(1, H_MID), W2, b2.reshape(1, H_MID), W3)

    p2 = sc_agg(y2, src_p, dst_p, zero_init)

    out = pl.pallas_call(
        _fin_body,
        out_shape=jax.ShapeDtypeStruct((N_NODES, D_OUT), jnp.float32),
    )(y2, p2, b3.reshape(1, H_MID), g3.reshape(1, H_MID),
      bt3.reshape(1, H_MID), W4, b4.reshape(1, D_OUT))
    return out
